# Initial kernel scaffold; baseline (speedup 1.0000x reference)
import jax
import jax.numpy as jnp
from jax.experimental import pallas as pl


def kernel(x, edge_index, W1, b1, W2, b2, W_ih_f, W_hh_f, b_ih_f, b_hh_f, W_ih_b, W_hh_b, b_ih_b, b_hh_b, att_W, att_b, W3, b3):
    raise NotImplementedError("write your pallas kernel here")


# speedup vs baseline: 21.9746x; 21.9746x over previous
import functools

import jax
import jax.numpy as jnp
from jax import lax
from jax.experimental import pallas as pl
from jax.experimental.pallas import tpu as pltpu
from jax.experimental.pallas import tpu_sc as plsc

N = 10000
D = 128
H = 16
LH = 32
OUT = 40
E = 320000

NC = 2
NS = 16
NW = NC * NS
CH = 128
KCH = 80
EPT = KCH * CH
EP = NW * EPT
NP = 10240
RPT = NP // NS
DUMMY = N

_mesh = plsc.VectorSubcoreMesh(core_axis_name="c", subcore_axis_name="s")
_sc_params = pltpu.CompilerParams(use_tc_tiling_on_sc=False)


@functools.partial(
    pl.kernel, mesh=_mesh, compiler_params=_sc_params,
    out_type=jax.ShapeDtypeStruct((NC, NP), jnp.float32),
    scratch_types=[
        pltpu.VMEM((KCH, CH), jnp.int32),
        pltpu.VMEM((CH,), jnp.float32),
        pltpu.VMEM((RPT,), jnp.float32),
        pltpu.VMEM_SHARED((NP,), jnp.float32),
        pltpu.SemaphoreType.DMA,
    ],
)
def _sc_degree(dst3_hbm, ones_hbm, zeros_hbm, out_hbm,
               dst_v, ones_v, zrow_v, acc_sh, sem):
    cid = lax.axis_index("c")
    sid = lax.axis_index("s")
    wid = sid * NC + cid
    pltpu.sync_copy(zeros_hbm, zrow_v)
    pltpu.sync_copy(zrow_v, acc_sh.at[pl.ds(sid * RPT, RPT)])
    pltpu.sync_copy(ones_hbm, ones_v)
    pltpu.sync_copy(dst3_hbm.at[wid], dst_v)
    plsc.subcore_barrier()

    def body(j, carry):
        pltpu.sync_copy(ones_v, acc_sh.at[dst_v.at[j]], add=True)
        return carry

    lax.fori_loop(0, KCH, body, 0)
    plsc.subcore_barrier()
    pltpu.sync_copy(acc_sh.at[pl.ds(sid * RPT, RPT)],
                    out_hbm.at[cid, pl.ds(sid * RPT, RPT)])


@functools.partial(
    pl.kernel, mesh=_mesh, compiler_params=_sc_params,
    out_type=jax.ShapeDtypeStruct((NC, NP, H), jnp.float32),
    scratch_types=[
        pltpu.VMEM((KCH, CH), jnp.int32),
        pltpu.VMEM((KCH, CH), jnp.int32),
        pltpu.VMEM((CH, H), jnp.float32),
        pltpu.VMEM((RPT, H), jnp.float32),
        pltpu.VMEM_SHARED((NP, H), jnp.float32),
        pltpu.SemaphoreType.DMA,
    ],
)
def _sc_scatter(y_hbm, src3_hbm, dst3_hbm, zeros_hbm, out_hbm,
                src_v, dst_v, rows_v, zrow_v, acc_sh, sem):
    cid = lax.axis_index("c")
    sid = lax.axis_index("s")
    wid = sid * NC + cid
    pltpu.sync_copy(zeros_hbm, zrow_v)
    pltpu.sync_copy(zrow_v, acc_sh.at[pl.ds(sid * RPT, RPT)])
    pltpu.sync_copy(src3_hbm.at[wid], src_v)
    pltpu.sync_copy(dst3_hbm.at[wid], dst_v)
    plsc.subcore_barrier()

    def body(j, carry):
        pltpu.async_copy(y_hbm.at[src_v.at[j]], rows_v, sem).wait()
        pltpu.sync_copy(rows_v, acc_sh.at[dst_v.at[j]], add=True)
        return carry

    lax.fori_loop(0, KCH, body, 0)
    plsc.subcore_barrier()
    pltpu.sync_copy(acc_sh.at[pl.ds(sid * RPT, RPT)],
                    out_hbm.at[cid, pl.ds(sid * RPT, RPT)])


BR = 1024
GRID = NP // BR


def _dinv_of(deg_ref):
    return lax.rsqrt(deg_ref[0, :] + deg_ref[1, :] + 1.0)[:, None]


def _tc_first(deg_ref, x_ref, w_ref, y_ref):
    xw = jnp.dot(x_ref[...], w_ref[...], preferred_element_type=jnp.float32)
    y_ref[...] = xw * _dinv_of(deg_ref)


def _tc_conv_epilogue(deg_ref, z_ref, y_ref, b_ref, w_ref, x1_ref, y2_ref):
    dinv = _dinv_of(deg_ref)
    x1 = jnp.maximum((z_ref[0] + z_ref[1] + y_ref[...]) * dinv + b_ref[0, :],
                     0.0)
    x1_ref[...] = x1
    y2_ref[...] = jnp.dot(x1, w_ref[...],
                          preferred_element_type=jnp.float32) * dinv


def _sig(v):
    return jax.nn.sigmoid(v)


def _cell(xt, h, c, wih_ref, whh_ref, b_ref, first):
    g = jnp.dot(xt, wih_ref[...], preferred_element_type=jnp.float32)
    if not first:
        g = g + jnp.dot(h, whh_ref[...], preferred_element_type=jnp.float32)
    g = g + b_ref[0, :]
    gi = _sig(g[:, 0 * LH:1 * LH])
    gf = _sig(g[:, 1 * LH:2 * LH])
    gg = jnp.tanh(g[:, 2 * LH:3 * LH])
    go = _sig(g[:, 3 * LH:4 * LH])
    c = gi * gg if first else gf * c + gi * gg
    h = go * jnp.tanh(c)
    return h, c


def _tc_jk(deg_ref, z_ref, y_ref, b2_ref, x1_ref,
           wihf_ref, whhf_ref, bf_ref, wihb_ref, whhb_ref, bb_ref,
           aa_ref, ab_ref, attb_ref, y3_ref):
    dinv = _dinv_of(deg_ref)
    x1 = x1_ref[...]
    x2 = jnp.maximum((z_ref[0] + z_ref[1] + y_ref[...]) * dinv + b2_ref[0, :],
                     0.0)
    h1f, c1 = _cell(x1, None, None, wihf_ref, whhf_ref, bf_ref, True)
    h2f, _ = _cell(x2, h1f, c1, wihf_ref, whhf_ref, bf_ref, False)
    h2b, cb = _cell(x2, None, None, wihb_ref, whhb_ref, bb_ref, True)
    h1b, _ = _cell(x1, h2b, cb, wihb_ref, whhb_ref, bb_ref, False)
    aA = aa_ref[0, :]
    aB = ab_ref[0, :]
    attb = attb_ref[0, 0]
    a0 = jnp.sum(h1f * aA, axis=1) + jnp.sum(h1b * aB, axis=1) + attb
    a1 = jnp.sum(h2f * aA, axis=1) + jnp.sum(h2b * aB, axis=1) + attb
    m = jnp.maximum(a0, a1)
    e0 = jnp.exp(a0 - m)
    e1 = jnp.exp(a1 - m)
    w0 = (e0 / (e0 + e1))[:, None]
    jk = w0 * x1 + (1.0 - w0) * x2
    y3_ref[...] = jk * dinv


def _tc_final(deg_ref, z_ref, y_ref, w_ref, b_ref, out_ref):
    dinv = _dinv_of(deg_ref)
    prop = (z_ref[0] + z_ref[1] + y_ref[...]) * dinv
    logits = jnp.dot(prop, w_ref[...],
                     preferred_element_type=jnp.float32) + b_ref[0, :]
    m = jnp.max(logits, axis=1, keepdims=True)
    s = jnp.log(jnp.sum(jnp.exp(logits - m), axis=1, keepdims=True))
    out_ref[...] = logits - m - s


def _row_spec(width):
    return pl.BlockSpec((BR, width), lambda i: (i, 0))


_DEG_SPEC = pl.BlockSpec((2, BR), lambda i: (0, i))
_Z_SPEC = pl.BlockSpec((2, BR, H), lambda i: (0, i, 0))


def _full(shape):
    return pl.BlockSpec(shape, lambda i: tuple(0 for _ in shape))


def _call_first(degp, xp, w1):
    return pl.pallas_call(
        _tc_first,
        grid=(GRID,),
        in_specs=[_DEG_SPEC, _row_spec(D), _full((D, H))],
        out_specs=_row_spec(H),
        out_shape=jax.ShapeDtypeStruct((NP, H), jnp.float32),
    )(degp, xp, w1)


def _call_conv_epilogue(degp, z, y, b, w):
    return pl.pallas_call(
        _tc_conv_epilogue,
        grid=(GRID,),
        in_specs=[_DEG_SPEC, _Z_SPEC, _row_spec(H), _full((1, H)),
                  _full((H, H))],
        out_specs=[_row_spec(H), _row_spec(H)],
        out_shape=[jax.ShapeDtypeStruct((NP, H), jnp.float32),
                   jax.ShapeDtypeStruct((NP, H), jnp.float32)],
    )(degp, z, y, b, w)


def _call_jk(degp, z, y2, b2, x1, wihf, whhf, bf, wihb, whhb, bb, aa, ab,
             attb):
    return pl.pallas_call(
        _tc_jk,
        grid=(GRID,),
        in_specs=[_DEG_SPEC, _Z_SPEC, _row_spec(H), _full((1, H)),
                  _row_spec(H),
                  _full((H, 4 * LH)), _full((LH, 4 * LH)), _full((1, 4 * LH)),
                  _full((H, 4 * LH)), _full((LH, 4 * LH)), _full((1, 4 * LH)),
                  _full((1, LH)), _full((1, LH)), _full((1, 1))],
        out_specs=_row_spec(H),
        out_shape=jax.ShapeDtypeStruct((NP, H), jnp.float32),
    )(degp, z, y2, b2, x1, wihf, whhf, bf, wihb, whhb, bb, aa, ab, attb)


def _call_final(degp, z, y3, w3, b3):
    return pl.pallas_call(
        _tc_final,
        grid=(GRID,),
        in_specs=[_DEG_SPEC, _Z_SPEC, _row_spec(H), _full((H, OUT)),
                  _full((1, OUT))],
        out_specs=_row_spec(OUT),
        out_shape=jax.ShapeDtypeStruct((NP, OUT), jnp.float32),
    )(degp, z, y3, w3, b3)


def kernel(x, edge_index, W1, b1, W2, b2, W_ih_f, W_hh_f, b_ih_f, b_hh_f,
           W_ih_b, W_hh_b, b_ih_b, b_hh_b, att_W, att_b, W3, b3):
    f32 = jnp.float32
    pad = jnp.full((EP - E,), DUMMY, jnp.int32)
    src3 = jnp.concatenate([edge_index[0], pad]).reshape(NW, KCH, CH)
    dst3 = jnp.concatenate([edge_index[1], pad]).reshape(NW, KCH, CH)
    xp = jnp.zeros((NP, D), f32).at[:N].set(x)
    ones_ch = jnp.ones((CH,), f32)
    zeros_r = jnp.zeros((RPT,), f32)
    zeros_rh = jnp.zeros((RPT, H), f32)

    degp = _sc_degree(dst3, ones_ch, zeros_r)
    y1 = _call_first(degp, xp, W1)
    z1 = _sc_scatter(y1, src3, dst3, zeros_rh)
    x1, y2 = _call_conv_epilogue(degp, z1, y1, b1.reshape(1, H), W2)
    z2 = _sc_scatter(y2, src3, dst3, zeros_rh)
    y3 = _call_jk(degp, z2, y2, b2.reshape(1, H), x1,
                  W_ih_f.T, W_hh_f.T, (b_ih_f + b_hh_f).reshape(1, 4 * LH),
                  W_ih_b.T, W_hh_b.T, (b_ih_b + b_hh_b).reshape(1, 4 * LH),
                  att_W[:LH, 0].reshape(1, LH), att_W[LH:, 0].reshape(1, LH),
                  (att_b[0] * jnp.ones((1, 1), f32)))
    z3 = _sc_scatter(y3, src3, dst3, zeros_rh)
    outp = _call_final(degp, z3, y3, W3, b3.reshape(1, OUT))
    return outp[:N]


# speedup vs baseline: 49.2155x; 2.2397x over previous
import functools

import jax
import jax.numpy as jnp
from jax import lax
from jax.experimental import pallas as pl
from jax.experimental.pallas import tpu as pltpu
from jax.experimental.pallas import tpu_sc as plsc

N = 10000
D = 128
H = 16
LH = 32
OUT = 40
E = 320000

NC = 2
NS = 16
NW = NC * NS
CH = 128
KCH = 80
EPT = KCH * CH
EP = NW * EPT
NP = 10240
RPT = NP // NS
DUMMY = N

_mesh = plsc.VectorSubcoreMesh(core_axis_name="c", subcore_axis_name="s")
_sc_params = pltpu.CompilerParams(use_tc_tiling_on_sc=False)


@functools.partial(
    pl.kernel, mesh=_mesh, compiler_params=_sc_params,
    out_type=jax.ShapeDtypeStruct((NC, NP), jnp.float32),
    scratch_types=[
        pltpu.VMEM((KCH, CH), jnp.int32),
        pltpu.VMEM((CH,), jnp.float32),
        pltpu.VMEM((RPT,), jnp.float32),
        pltpu.VMEM_SHARED((NP,), jnp.float32),
        pltpu.SemaphoreType.DMA,
    ],
)
def _sc_degree(dst3_hbm, ones_hbm, zeros_hbm, out_hbm,
               dst_v, ones_v, zrow_v, acc_sh, sem):
    cid = lax.axis_index("c")
    sid = lax.axis_index("s")
    wid = sid * NC + cid
    pltpu.sync_copy(zeros_hbm, zrow_v)
    pltpu.sync_copy(zrow_v, acc_sh.at[pl.ds(sid * RPT, RPT)])
    pltpu.sync_copy(ones_hbm, ones_v)
    pltpu.sync_copy(dst3_hbm.at[wid], dst_v)
    plsc.subcore_barrier()

    def body(j, carry):
        pltpu.sync_copy(ones_v, acc_sh.at[dst_v.at[j]], add=True)
        return carry

    lax.fori_loop(0, KCH, body, 0)
    plsc.subcore_barrier()
    pltpu.sync_copy(acc_sh.at[pl.ds(sid * RPT, RPT)],
                    out_hbm.at[cid, pl.ds(sid * RPT, RPT)])


NB = 8
NGRP = KCH // NB


@functools.partial(
    pl.kernel, mesh=_mesh, compiler_params=_sc_params,
    out_type=jax.ShapeDtypeStruct((NC, NP, H), jnp.float32),
    scratch_types=[
        pltpu.VMEM((KCH, CH), jnp.int32),
        pltpu.VMEM((KCH, CH), jnp.int32),
        pltpu.VMEM((NB, CH, H), jnp.float32),
        pltpu.VMEM((RPT, H), jnp.float32),
        pltpu.VMEM_SHARED((NP, H), jnp.float32),
        pltpu.SemaphoreType.DMA((NB,)),
        pltpu.SemaphoreType.DMA((NB,)),
    ],
)
def _sc_scatter(y_hbm, src3_hbm, dst3_hbm, zeros_hbm, out_hbm,
                src_v, dst_v, rows_v, zrow_v, acc_sh, gsem, ssem):
    cid = lax.axis_index("c")
    sid = lax.axis_index("s")
    wid = sid * NC + cid
    pltpu.sync_copy(zeros_hbm, zrow_v)
    pltpu.sync_copy(zrow_v, acc_sh.at[pl.ds(sid * RPT, RPT)])
    pltpu.sync_copy(src3_hbm.at[wid], src_v)
    pltpu.sync_copy(dst3_hbm.at[wid], dst_v)
    plsc.subcore_barrier()

    def _gather_start(j, b):
        pltpu.async_copy(y_hbm.at[src_v.at[j]], rows_v.at[b], gsem.at[b])

    def _gather_wait(j, b):
        pltpu.make_async_copy(y_hbm.at[src_v.at[j]], rows_v.at[b],
                              gsem.at[b]).wait()

    def _scatter_start(j, b):
        pltpu.async_copy(rows_v.at[b], acc_sh.at[dst_v.at[j]], ssem.at[b],
                         add=True)

    def _scatter_wait(j, b):
        pltpu.make_async_copy(rows_v.at[b], acc_sh.at[dst_v.at[j]],
                              ssem.at[b]).wait()

    for b in range(NB):
        _gather_start(b, b)

    def group(g, carry):
        for b in range(NB):
            j = g * NB + b
            _gather_wait(j, b)
            _scatter_start(j, b)

            @pl.when(g < NGRP - 1)
            def _():
                _scatter_wait(j, b)
                _gather_start(j + NB, b)

        return carry

    lax.fori_loop(0, NGRP, group, 0)
    for b in range(NB):
        _scatter_wait((NGRP - 1) * NB + b, b)
    plsc.subcore_barrier()
    pltpu.sync_copy(acc_sh.at[pl.ds(sid * RPT, RPT)],
                    out_hbm.at[cid, pl.ds(sid * RPT, RPT)])


BR = 1024
GRID = NP // BR


def _dinv_of(deg_ref):
    return lax.rsqrt(deg_ref[0, :] + deg_ref[1, :] + 1.0)[:, None]


def _tc_first(deg_ref, x_ref, w_ref, y_ref):
    xw = jnp.dot(x_ref[...], w_ref[...], preferred_element_type=jnp.float32)
    y_ref[...] = xw * _dinv_of(deg_ref)


def _tc_conv_epilogue(deg_ref, z_ref, y_ref, b_ref, w_ref, x1_ref, y2_ref):
    dinv = _dinv_of(deg_ref)
    x1 = jnp.maximum((z_ref[0] + z_ref[1] + y_ref[...]) * dinv + b_ref[0, :],
                     0.0)
    x1_ref[...] = x1
    y2_ref[...] = jnp.dot(x1, w_ref[...],
                          preferred_element_type=jnp.float32) * dinv


def _sig(v):
    return jax.nn.sigmoid(v)


def _cell(xt, h, c, wih_ref, whh_ref, b_ref, first):
    g = jnp.dot(xt, wih_ref[...], preferred_element_type=jnp.float32)
    if not first:
        g = g + jnp.dot(h, whh_ref[...], preferred_element_type=jnp.float32)
    g = g + b_ref[0, :]
    gi = _sig(g[:, 0 * LH:1 * LH])
    gf = _sig(g[:, 1 * LH:2 * LH])
    gg = jnp.tanh(g[:, 2 * LH:3 * LH])
    go = _sig(g[:, 3 * LH:4 * LH])
    c = gi * gg if first else gf * c + gi * gg
    h = go * jnp.tanh(c)
    return h, c


def _tc_jk(deg_ref, z_ref, y_ref, b2_ref, x1_ref,
           wihf_ref, whhf_ref, bf_ref, wihb_ref, whhb_ref, bb_ref,
           aa_ref, ab_ref, attb_ref, y3_ref):
    dinv = _dinv_of(deg_ref)
    x1 = x1_ref[...]
    x2 = jnp.maximum((z_ref[0] + z_ref[1] + y_ref[...]) * dinv + b2_ref[0, :],
                     0.0)
    h1f, c1 = _cell(x1, None, None, wihf_ref, whhf_ref, bf_ref, True)
    h2f, _ = _cell(x2, h1f, c1, wihf_ref, whhf_ref, bf_ref, False)
    h2b, cb = _cell(x2, None, None, wihb_ref, whhb_ref, bb_ref, True)
    h1b, _ = _cell(x1, h2b, cb, wihb_ref, whhb_ref, bb_ref, False)
    aA = aa_ref[0, :]
    aB = ab_ref[0, :]
    attb = attb_ref[0, 0]
    a0 = jnp.sum(h1f * aA, axis=1) + jnp.sum(h1b * aB, axis=1) + attb
    a1 = jnp.sum(h2f * aA, axis=1) + jnp.sum(h2b * aB, axis=1) + attb
    m = jnp.maximum(a0, a1)
    e0 = jnp.exp(a0 - m)
    e1 = jnp.exp(a1 - m)
    w0 = (e0 / (e0 + e1))[:, None]
    jk = w0 * x1 + (1.0 - w0) * x2
    y3_ref[...] = jk * dinv


def _tc_final(deg_ref, z_ref, y_ref, w_ref, b_ref, out_ref):
    dinv = _dinv_of(deg_ref)
    prop = (z_ref[0] + z_ref[1] + y_ref[...]) * dinv
    logits = jnp.dot(prop, w_ref[...],
                     preferred_element_type=jnp.float32) + b_ref[0, :]
    m = jnp.max(logits, axis=1, keepdims=True)
    s = jnp.log(jnp.sum(jnp.exp(logits - m), axis=1, keepdims=True))
    out_ref[...] = logits - m - s


def _row_spec(width):
    return pl.BlockSpec((BR, width), lambda i: (i, 0))


_DEG_SPEC = pl.BlockSpec((2, BR), lambda i: (0, i))
_Z_SPEC = pl.BlockSpec((2, BR, H), lambda i: (0, i, 0))


def _full(shape):
    return pl.BlockSpec(shape, lambda i: tuple(0 for _ in shape))


def _call_first(degp, xp, w1):
    return pl.pallas_call(
        _tc_first,
        grid=(GRID,),
        in_specs=[_DEG_SPEC, _row_spec(D), _full((D, H))],
        out_specs=_row_spec(H),
        out_shape=jax.ShapeDtypeStruct((NP, H), jnp.float32),
    )(degp, xp, w1)


def _call_conv_epilogue(degp, z, y, b, w):
    return pl.pallas_call(
        _tc_conv_epilogue,
        grid=(GRID,),
        in_specs=[_DEG_SPEC, _Z_SPEC, _row_spec(H), _full((1, H)),
                  _full((H, H))],
        out_specs=[_row_spec(H), _row_spec(H)],
        out_shape=[jax.ShapeDtypeStruct((NP, H), jnp.float32),
                   jax.ShapeDtypeStruct((NP, H), jnp.float32)],
    )(degp, z, y, b, w)


def _call_jk(degp, z, y2, b2, x1, wihf, whhf, bf, wihb, whhb, bb, aa, ab,
             attb):
    return pl.pallas_call(
        _tc_jk,
        grid=(GRID,),
        in_specs=[_DEG_SPEC, _Z_SPEC, _row_spec(H), _full((1, H)),
                  _row_spec(H),
                  _full((H, 4 * LH)), _full((LH, 4 * LH)), _full((1, 4 * LH)),
                  _full((H, 4 * LH)), _full((LH, 4 * LH)), _full((1, 4 * LH)),
                  _full((1, LH)), _full((1, LH)), _full((1, 1))],
        out_specs=_row_spec(H),
        out_shape=jax.ShapeDtypeStruct((NP, H), jnp.float32),
    )(degp, z, y2, b2, x1, wihf, whhf, bf, wihb, whhb, bb, aa, ab, attb)


def _call_final(degp, z, y3, w3, b3):
    return pl.pallas_call(
        _tc_final,
        grid=(GRID,),
        in_specs=[_DEG_SPEC, _Z_SPEC, _row_spec(H), _full((H, OUT)),
                  _full((1, OUT))],
        out_specs=_row_spec(OUT),
        out_shape=jax.ShapeDtypeStruct((NP, OUT), jnp.float32),
    )(degp, z, y3, w3, b3)


def kernel(x, edge_index, W1, b1, W2, b2, W_ih_f, W_hh_f, b_ih_f, b_hh_f,
           W_ih_b, W_hh_b, b_ih_b, b_hh_b, att_W, att_b, W3, b3):
    f32 = jnp.float32
    pad = DUMMY + (jnp.arange(EP - E, dtype=jnp.int32) % (NP - N))
    src3 = jnp.concatenate([edge_index[0], pad]).reshape(NW, KCH, CH)
    dst3 = jnp.concatenate([edge_index[1], pad]).reshape(NW, KCH, CH)
    xp = jnp.zeros((NP, D), f32).at[:N].set(x)
    ones_ch = jnp.ones((CH,), f32)
    zeros_r = jnp.zeros((RPT,), f32)
    zeros_rh = jnp.zeros((RPT, H), f32)

    degp = _sc_degree(dst3, ones_ch, zeros_r)
    y1 = _call_first(degp, xp, W1)
    z1 = _sc_scatter(y1, src3, dst3, zeros_rh)
    x1, y2 = _call_conv_epilogue(degp, z1, y1, b1.reshape(1, H), W2)
    z2 = _sc_scatter(y2, src3, dst3, zeros_rh)
    y3 = _call_jk(degp, z2, y2, b2.reshape(1, H), x1,
                  W_ih_f.T, W_hh_f.T, (b_ih_f + b_hh_f).reshape(1, 4 * LH),
                  W_ih_b.T, W_hh_b.T, (b_ih_b + b_hh_b).reshape(1, 4 * LH),
                  att_W[:LH, 0].reshape(1, LH), att_W[LH:, 0].reshape(1, LH),
                  (att_b[0] * jnp.ones((1, 1), f32)))
    z3 = _sc_scatter(y3, src3, dst3, zeros_rh)
    outp = _call_final(degp, z3, y3, W3, b3.reshape(1, OUT))
    return outp[:N]


# speedup vs baseline: 51.8138x; 1.0528x over previous
import functools

import jax
import jax.numpy as jnp
from jax import lax
from jax.experimental import pallas as pl
from jax.experimental.pallas import tpu as pltpu
from jax.experimental.pallas import tpu_sc as plsc

N = 10000
D = 128
H = 16
LH = 32
OUT = 40
E = 320000

NC = 2
NS = 16
NW = NC * NS
CH = 128
KCH = 80
EPT = KCH * CH
EP = NW * EPT
NP = 10240
RPT = NP // NS
DUMMY = N

_mesh = plsc.VectorSubcoreMesh(core_axis_name="c", subcore_axis_name="s")
_sc_params = pltpu.CompilerParams(use_tc_tiling_on_sc=False)


@functools.partial(
    pl.kernel, mesh=_mesh, compiler_params=_sc_params,
    out_type=jax.ShapeDtypeStruct((NC, NP), jnp.float32),
    scratch_types=[
        pltpu.VMEM((KCH, CH), jnp.int32),
        pltpu.VMEM((CH,), jnp.float32),
        pltpu.VMEM((RPT,), jnp.float32),
        pltpu.VMEM_SHARED((NP,), jnp.float32),
        pltpu.SemaphoreType.DMA,
    ],
)
def _sc_degree(dst3_hbm, ones_hbm, zeros_hbm, out_hbm,
               dst_v, ones_v, zrow_v, acc_sh, sem):
    cid = lax.axis_index("c")
    sid = lax.axis_index("s")
    wid = sid * NC + cid
    pltpu.sync_copy(zeros_hbm, zrow_v)
    pltpu.sync_copy(zrow_v, acc_sh.at[pl.ds(sid * RPT, RPT)])
    pltpu.sync_copy(ones_hbm, ones_v)
    pltpu.sync_copy(dst3_hbm.at[wid], dst_v)
    plsc.subcore_barrier()

    def body(j, carry):
        pltpu.sync_copy(ones_v, acc_sh.at[dst_v.at[j]], add=True)
        return carry

    lax.fori_loop(0, KCH, body, 0)
    plsc.subcore_barrier()
    pltpu.sync_copy(acc_sh.at[pl.ds(sid * RPT, RPT)],
                    out_hbm.at[cid, pl.ds(sid * RPT, RPT)])


NB = 8
NGRP = KCH // NB


@functools.partial(
    pl.kernel, mesh=_mesh, compiler_params=_sc_params,
    out_type=jax.ShapeDtypeStruct((NC, NP, H), jnp.float32),
    scratch_types=[
        pltpu.VMEM((KCH, CH), jnp.int32),
        pltpu.VMEM((KCH, CH), jnp.int32),
        pltpu.VMEM((NB, CH, H), jnp.float32),
        pltpu.VMEM((RPT, H), jnp.float32),
        pltpu.VMEM_SHARED((NP, H), jnp.float32),
        pltpu.SemaphoreType.DMA((NB,)),
        pltpu.SemaphoreType.DMA((NB,)),
    ],
)
def _sc_scatter(y_hbm, src3_hbm, dst3_hbm, zeros_hbm, out_hbm,
                src_v, dst_v, rows_v, zrow_v, acc_sh, gsem, ssem):
    cid = lax.axis_index("c")
    sid = lax.axis_index("s")
    wid = sid * NC + cid
    pltpu.sync_copy(zeros_hbm, zrow_v)
    pltpu.sync_copy(zrow_v, acc_sh.at[pl.ds(sid * RPT, RPT)])
    pltpu.sync_copy(src3_hbm.at[wid], src_v)
    pltpu.sync_copy(dst3_hbm.at[wid], dst_v)
    plsc.subcore_barrier()

    def _gather_start(j, b):
        pltpu.async_copy(y_hbm.at[src_v.at[j]], rows_v.at[b], gsem.at[b])

    def _gather_wait(j, b):
        pltpu.make_async_copy(y_hbm.at[src_v.at[j]], rows_v.at[b],
                              gsem.at[b]).wait()

    def _scatter_start(j, b):
        pltpu.async_copy(rows_v.at[b], acc_sh.at[dst_v.at[j]], ssem.at[b],
                         add=True)

    def _scatter_wait(j, b):
        pltpu.make_async_copy(rows_v.at[b], acc_sh.at[dst_v.at[j]],
                              ssem.at[b]).wait()

    for b in range(NB):
        _gather_start(b, b)

    def group(g, carry):
        for b in range(NB):
            j = g * NB + b
            _gather_wait(j, b)
            _scatter_start(j, b)

            @pl.when(g < NGRP - 1)
            def _():
                _scatter_wait(j, b)
                _gather_start(j + NB, b)

        return carry

    lax.fori_loop(0, NGRP, group, 0)
    for b in range(NB):
        _scatter_wait((NGRP - 1) * NB + b, b)
    plsc.subcore_barrier()
    pltpu.sync_copy(acc_sh.at[pl.ds(sid * RPT, RPT)],
                    out_hbm.at[cid, pl.ds(sid * RPT, RPT)])


BR = 1024
GRID = NP // BR


def _dinv_of(deg_ref):
    return lax.rsqrt(deg_ref[0, :] + deg_ref[1, :] + 1.0)[:, None]


def _tc_first(deg_ref, x_ref, w_ref, y_ref):
    xw = jnp.dot(x_ref[...], w_ref[...], preferred_element_type=jnp.float32)
    y_ref[...] = xw * _dinv_of(deg_ref)


def _tc_conv_epilogue(deg_ref, z_ref, y_ref, b_ref, w_ref, x1_ref, y2_ref):
    dinv = _dinv_of(deg_ref)
    x1 = jnp.maximum((z_ref[0] + z_ref[1] + y_ref[...]) * dinv + b_ref[0, :],
                     0.0)
    x1_ref[...] = x1
    y2_ref[...] = jnp.dot(x1, w_ref[...],
                          preferred_element_type=jnp.float32) * dinv


def _sig(v):
    return jax.nn.sigmoid(v)


def _dot(a, b_ref):
    return jnp.dot(a, b_ref[...], preferred_element_type=jnp.float32)


def _lstm_dir(xa, xb, w, y3_ref=None):
    wxi, wxf, wxg, wxo, whi, whf, whg, who, bi, bf, bg, bo = w
    gi = _sig(_dot(xa, wxi) + bi[0, :])
    gg = jnp.tanh(_dot(xa, wxg) + bg[0, :])
    go = _sig(_dot(xa, wxo) + bo[0, :])
    c1 = gi * gg
    h1 = go * jnp.tanh(c1)
    gi = _sig(_dot(xb, wxi) + _dot(h1, whi) + bi[0, :])
    gf = _sig(_dot(xb, wxf) + _dot(h1, whf) + bf[0, :])
    gg = jnp.tanh(_dot(xb, wxg) + _dot(h1, whg) + bg[0, :])
    go = _sig(_dot(xb, wxo) + _dot(h1, who) + bo[0, :])
    c2 = gf * c1 + gi * gg
    h2 = go * jnp.tanh(c2)
    return h1, h2


def _tc_jk(deg_ref, z_ref, y_ref, b2_ref, x1_ref,
           wxi_f, wxf_f, wxg_f, wxo_f, whi_f, whf_f, whg_f, who_f,
           bi_f, bf_f, bg_f, bo_f,
           wxi_b, wxf_b, wxg_b, wxo_b, whi_b, whf_b, whg_b, who_b,
           bi_b, bf_b, bg_b, bo_b,
           aa_ref, ab_ref, y3_ref):
    dinv = _dinv_of(deg_ref)
    x1 = x1_ref[...]
    x2 = jnp.maximum((z_ref[0] + z_ref[1] + y_ref[...]) * dinv + b2_ref[0, :],
                     0.0)
    h1f, h2f = _lstm_dir(x1, x2, (wxi_f, wxf_f, wxg_f, wxo_f,
                                  whi_f, whf_f, whg_f, who_f,
                                  bi_f, bf_f, bg_f, bo_f))
    h2b, h1b = _lstm_dir(x2, x1, (wxi_b, wxf_b, wxg_b, wxo_b,
                                  whi_b, whf_b, whg_b, who_b,
                                  bi_b, bf_b, bg_b, bo_b))
    delta = jnp.sum((h1f - h2f) * aa_ref[0, :] + (h1b - h2b) * ab_ref[0, :],
                    axis=1)
    w0 = _sig(delta)[:, None]
    jk = w0 * x1 + (1.0 - w0) * x2
    y3_ref[...] = jk * dinv


def _tc_final(deg_ref, z_ref, y_ref, w_ref, b_ref, out_ref):
    dinv = _dinv_of(deg_ref)
    prop = (z_ref[0] + z_ref[1] + y_ref[...]) * dinv
    logits = jnp.dot(prop, w_ref[...],
                     preferred_element_type=jnp.float32) + b_ref[0, :]
    m = jnp.max(logits, axis=1, keepdims=True)
    s = jnp.log(jnp.sum(jnp.exp(logits - m), axis=1, keepdims=True))
    out_ref[...] = logits - m - s


def _row_spec(width):
    return pl.BlockSpec((BR, width), lambda i: (i, 0))


_DEG_SPEC = pl.BlockSpec((2, BR), lambda i: (0, i))
_Z_SPEC = pl.BlockSpec((2, BR, H), lambda i: (0, i, 0))


def _full(shape):
    return pl.BlockSpec(shape, lambda i: tuple(0 for _ in shape))


def _call_first(degp, xp, w1):
    return pl.pallas_call(
        _tc_first,
        grid=(GRID,),
        in_specs=[_DEG_SPEC, _row_spec(D), _full((D, H))],
        out_specs=_row_spec(H),
        out_shape=jax.ShapeDtypeStruct((NP, H), jnp.float32),
    )(degp, xp, w1)


def _call_conv_epilogue(degp, z, y, b, w):
    return pl.pallas_call(
        _tc_conv_epilogue,
        grid=(GRID,),
        in_specs=[_DEG_SPEC, _Z_SPEC, _row_spec(H), _full((1, H)),
                  _full((H, H))],
        out_specs=[_row_spec(H), _row_spec(H)],
        out_shape=[jax.ShapeDtypeStruct((NP, H), jnp.float32),
                   jax.ShapeDtypeStruct((NP, H), jnp.float32)],
    )(degp, z, y, b, w)


def _call_jk(degp, z, y2, b2, x1, *lstm_args):
    wspecs = ([_full((H, LH))] * 4 + [_full((LH, LH))] * 4 +
              [_full((1, LH))] * 4) * 2 + [_full((1, LH))] * 2
    return pl.pallas_call(
        _tc_jk,
        grid=(GRID,),
        in_specs=[_DEG_SPEC, _Z_SPEC, _row_spec(H), _full((1, H)),
                  _row_spec(H)] + wspecs,
        out_specs=_row_spec(H),
        out_shape=jax.ShapeDtypeStruct((NP, H), jnp.float32),
    )(degp, z, y2, b2, x1, *lstm_args)


def _call_final(degp, z, y3, w3, b3):
    return pl.pallas_call(
        _tc_final,
        grid=(GRID,),
        in_specs=[_DEG_SPEC, _Z_SPEC, _row_spec(H), _full((H, OUT)),
                  _full((1, OUT))],
        out_specs=_row_spec(OUT),
        out_shape=jax.ShapeDtypeStruct((NP, OUT), jnp.float32),
    )(degp, z, y3, w3, b3)


def kernel(x, edge_index, W1, b1, W2, b2, W_ih_f, W_hh_f, b_ih_f, b_hh_f,
           W_ih_b, W_hh_b, b_ih_b, b_hh_b, att_W, att_b, W3, b3):
    f32 = jnp.float32
    pad = DUMMY + (jnp.arange(EP - E, dtype=jnp.int32) % (NP - N))
    src3 = jnp.concatenate([edge_index[0], pad]).reshape(NW, KCH, CH)
    dst3 = jnp.concatenate([edge_index[1], pad]).reshape(NW, KCH, CH)
    ones_ch = jnp.ones((CH,), f32)
    zeros_r = jnp.zeros((RPT,), f32)
    zeros_rh = jnp.zeros((RPT, H), f32)

    def gate_args(wih, whh, bih, bhh):
        b = (bih + bhh)
        out = []
        for g in range(4):
            out.append(wih[g * LH:(g + 1) * LH].T)
        for g in range(4):
            out.append(whh[g * LH:(g + 1) * LH].T)
        for g in range(4):
            out.append(b[g * LH:(g + 1) * LH].reshape(1, LH))
        return out

    xp = jnp.zeros((NP, D), f32).at[:N].set(x)
    degp = _sc_degree(dst3, ones_ch, zeros_r)
    y1 = _call_first(degp, xp, W1)
    z1 = _sc_scatter(y1, src3, dst3, zeros_rh)
    x1, y2 = _call_conv_epilogue(degp, z1, y1, b1.reshape(1, H), W2)
    z2 = _sc_scatter(y2, src3, dst3, zeros_rh)
    y3 = _call_jk(degp, z2, y2, b2.reshape(1, H), x1,
                  *gate_args(W_ih_f, W_hh_f, b_ih_f, b_hh_f),
                  *gate_args(W_ih_b, W_hh_b, b_ih_b, b_hh_b),
                  att_W[:LH, 0].reshape(1, LH), att_W[LH:, 0].reshape(1, LH))
    z3 = _sc_scatter(y3, src3, dst3, zeros_rh)
    outp = _call_final(degp, z3, y3, W3, b3.reshape(1, OUT))
    return outp[:N]


# speedup vs baseline: 53.6464x; 1.0354x over previous
import functools

import jax
import jax.numpy as jnp
from jax import lax
from jax.experimental import pallas as pl
from jax.experimental.pallas import tpu as pltpu
from jax.experimental.pallas import tpu_sc as plsc

N = 10000
D = 128
H = 16
LH = 32
OUT = 40
E = 320000

NC = 2
NS = 16
NW = NC * NS
CH = 128
KCH = 80
EPT = KCH * CH
EP = NW * EPT
NP = 10240
RPT = NP // NS
DUMMY = N

_mesh = plsc.VectorSubcoreMesh(core_axis_name="c", subcore_axis_name="s")
_sc_params = pltpu.CompilerParams(use_tc_tiling_on_sc=False)


@functools.partial(
    pl.kernel, mesh=_mesh, compiler_params=_sc_params,
    out_type=jax.ShapeDtypeStruct((NC, NP), jnp.float32),
    scratch_types=[
        pltpu.VMEM((KCH, CH), jnp.int32),
        pltpu.VMEM((CH,), jnp.float32),
        pltpu.VMEM((RPT,), jnp.float32),
        pltpu.VMEM_SHARED((NP,), jnp.float32),
        pltpu.SemaphoreType.DMA,
    ],
)
def _sc_degree(dst3_hbm, ones_hbm, zeros_hbm, out_hbm,
               dst_v, ones_v, zrow_v, acc_sh, sem):
    cid = lax.axis_index("c")
    sid = lax.axis_index("s")
    wid = sid * NC + cid
    pltpu.sync_copy(zeros_hbm, zrow_v)
    pltpu.sync_copy(zrow_v, acc_sh.at[pl.ds(sid * RPT, RPT)])
    pltpu.sync_copy(ones_hbm, ones_v)
    pltpu.sync_copy(dst3_hbm.at[wid], dst_v)
    plsc.subcore_barrier()

    def body(j, carry):
        pltpu.sync_copy(ones_v, acc_sh.at[dst_v.at[j]], add=True)
        return carry

    lax.fori_loop(0, KCH, body, 0)
    plsc.subcore_barrier()
    pltpu.sync_copy(acc_sh.at[pl.ds(sid * RPT, RPT)],
                    out_hbm.at[cid, pl.ds(sid * RPT, RPT)])


NB = 8
NGRP = KCH // NB


@functools.partial(
    pl.kernel, mesh=_mesh, compiler_params=_sc_params,
    out_type=jax.ShapeDtypeStruct((NC, NP, H), jnp.float32),
    scratch_types=[
        pltpu.VMEM((KCH, CH), jnp.int32),
        pltpu.VMEM((KCH, CH), jnp.int32),
        pltpu.VMEM((NB, CH, H), jnp.float32),
        pltpu.VMEM((RPT, H), jnp.float32),
        pltpu.VMEM_SHARED((NP, H), jnp.float32),
        pltpu.SemaphoreType.DMA((NB,)),
        pltpu.SemaphoreType.DMA((NB,)),
    ],
)
def _sc_scatter(y_hbm, src3_hbm, dst3_hbm, zeros_hbm, out_hbm,
                src_v, dst_v, rows_v, zrow_v, acc_sh, gsem, ssem):
    cid = lax.axis_index("c")
    sid = lax.axis_index("s")
    wid = sid * NC + cid
    pltpu.sync_copy(zeros_hbm, zrow_v)
    pltpu.sync_copy(zrow_v, acc_sh.at[pl.ds(sid * RPT, RPT)])
    pltpu.sync_copy(src3_hbm.at[wid], src_v)
    pltpu.sync_copy(dst3_hbm.at[wid], dst_v)
    plsc.subcore_barrier()

    def _gather_start(j, b):
        pltpu.async_copy(y_hbm.at[src_v.at[j]], rows_v.at[b], gsem.at[b])

    def _gather_wait(j, b):
        pltpu.make_async_copy(y_hbm.at[src_v.at[j]], rows_v.at[b],
                              gsem.at[b]).wait()

    def _scatter_start(j, b):
        pltpu.async_copy(rows_v.at[b], acc_sh.at[dst_v.at[j]], ssem.at[b],
                         add=True)

    def _scatter_wait(j, b):
        pltpu.make_async_copy(rows_v.at[b], acc_sh.at[dst_v.at[j]],
                              ssem.at[b]).wait()

    for b in range(NB):
        _gather_start(b, b)

    def group(g, carry):
        for b in range(NB):
            j = g * NB + b
            _gather_wait(j, b)
            _scatter_start(j, b)

            @pl.when(g < NGRP - 1)
            def _():
                _scatter_wait(j, b)
                _gather_start(j + NB, b)

        return carry

    lax.fori_loop(0, NGRP, group, 0)
    for b in range(NB):
        _scatter_wait((NGRP - 1) * NB + b, b)
    plsc.subcore_barrier()
    pltpu.sync_copy(acc_sh.at[pl.ds(sid * RPT, RPT)],
                    out_hbm.at[cid, pl.ds(sid * RPT, RPT)])


BR = 2048
GRID = NP // BR


def _dinv_of(deg_ref):
    return lax.rsqrt(deg_ref[0, :] + deg_ref[1, :] + 1.0)[:, None]


def _tc_first(deg_ref, x_ref, w_ref, y_ref):
    xw = jnp.dot(x_ref[...], w_ref[...], preferred_element_type=jnp.float32)
    y_ref[...] = xw * _dinv_of(deg_ref)


def _tc_conv_epilogue(deg_ref, z_ref, y_ref, b_ref, w_ref, x1_ref, y2_ref):
    dinv = _dinv_of(deg_ref)
    x1 = jnp.maximum((z_ref[0] + z_ref[1] + y_ref[...]) * dinv + b_ref[0, :],
                     0.0)
    x1_ref[...] = x1
    y2_ref[...] = jnp.dot(x1, w_ref[...],
                          preferred_element_type=jnp.float32) * dinv


def _sig(v):
    return jax.nn.sigmoid(v)


def _dot(a, b_ref):
    return jnp.dot(a, b_ref[...], preferred_element_type=jnp.float32)


def _lstm_dir(xa, xb, w, y3_ref=None):
    wxi, wxf, wxg, wxo, whi, whf, whg, who, bi, bf, bg, bo = w
    gi = _sig(_dot(xa, wxi) + bi[0, :])
    gg = jnp.tanh(_dot(xa, wxg) + bg[0, :])
    go = _sig(_dot(xa, wxo) + bo[0, :])
    c1 = gi * gg
    h1 = go * jnp.tanh(c1)
    gi = _sig(_dot(xb, wxi) + _dot(h1, whi) + bi[0, :])
    gf = _sig(_dot(xb, wxf) + _dot(h1, whf) + bf[0, :])
    gg = jnp.tanh(_dot(xb, wxg) + _dot(h1, whg) + bg[0, :])
    go = _sig(_dot(xb, wxo) + _dot(h1, who) + bo[0, :])
    c2 = gf * c1 + gi * gg
    h2 = go * jnp.tanh(c2)
    return h1, h2


def _tc_jk(deg_ref, z_ref, y_ref, b2_ref, x1_ref,
           wxi_f, wxf_f, wxg_f, wxo_f, whi_f, whf_f, whg_f, who_f,
           bi_f, bf_f, bg_f, bo_f,
           wxi_b, wxf_b, wxg_b, wxo_b, whi_b, whf_b, whg_b, who_b,
           bi_b, bf_b, bg_b, bo_b,
           aa_ref, ab_ref, y3_ref):
    dinv = _dinv_of(deg_ref)
    x1 = x1_ref[...]
    x2 = jnp.maximum((z_ref[0] + z_ref[1] + y_ref[...]) * dinv + b2_ref[0, :],
                     0.0)
    h1f, h2f = _lstm_dir(x1, x2, (wxi_f, wxf_f, wxg_f, wxo_f,
                                  whi_f, whf_f, whg_f, who_f,
                                  bi_f, bf_f, bg_f, bo_f))
    h2b, h1b = _lstm_dir(x2, x1, (wxi_b, wxf_b, wxg_b, wxo_b,
                                  whi_b, whf_b, whg_b, who_b,
                                  bi_b, bf_b, bg_b, bo_b))
    delta = jnp.sum((h1f - h2f) * aa_ref[0, :] + (h1b - h2b) * ab_ref[0, :],
                    axis=1)
    w0 = _sig(delta)[:, None]
    jk = w0 * x1 + (1.0 - w0) * x2
    y3_ref[...] = jk * dinv


def _tc_final(deg_ref, z_ref, y_ref, w_ref, b_ref, out_ref):
    dinv = _dinv_of(deg_ref)
    prop = (z_ref[0] + z_ref[1] + y_ref[...]) * dinv
    logits = jnp.dot(prop, w_ref[...],
                     preferred_element_type=jnp.float32) + b_ref[0, :]
    m = jnp.max(logits, axis=1, keepdims=True)
    s = jnp.log(jnp.sum(jnp.exp(logits - m), axis=1, keepdims=True))
    out_ref[...] = logits - m - s


def _row_spec(width):
    return pl.BlockSpec((BR, width), lambda i: (i, 0))


_DEG_SPEC = pl.BlockSpec((2, BR), lambda i: (0, i))
_Z_SPEC = pl.BlockSpec((2, BR, H), lambda i: (0, i, 0))


def _full(shape):
    return pl.BlockSpec(shape, lambda i: tuple(0 for _ in shape))


def _call_first(degp, xp, w1):
    return pl.pallas_call(
        _tc_first,
        grid=(GRID,),
        in_specs=[_DEG_SPEC, _row_spec(D), _full((D, H))],
        out_specs=_row_spec(H),
        out_shape=jax.ShapeDtypeStruct((NP, H), jnp.float32),
    )(degp, xp, w1)


def _call_conv_epilogue(degp, z, y, b, w):
    return pl.pallas_call(
        _tc_conv_epilogue,
        grid=(GRID,),
        in_specs=[_DEG_SPEC, _Z_SPEC, _row_spec(H), _full((1, H)),
                  _full((H, H))],
        out_specs=[_row_spec(H), _row_spec(H)],
        out_shape=[jax.ShapeDtypeStruct((NP, H), jnp.float32),
                   jax.ShapeDtypeStruct((NP, H), jnp.float32)],
    )(degp, z, y, b, w)


def _call_jk(degp, z, y2, b2, x1, *lstm_args):
    wspecs = ([_full((H, LH))] * 4 + [_full((LH, LH))] * 4 +
              [_full((1, LH))] * 4) * 2 + [_full((1, LH))] * 2
    return pl.pallas_call(
        _tc_jk,
        grid=(GRID,),
        in_specs=[_DEG_SPEC, _Z_SPEC, _row_spec(H), _full((1, H)),
                  _row_spec(H)] + wspecs,
        out_specs=_row_spec(H),
        out_shape=jax.ShapeDtypeStruct((NP, H), jnp.float32),
    )(degp, z, y2, b2, x1, *lstm_args)


def _call_final(degp, z, y3, w3, b3):
    return pl.pallas_call(
        _tc_final,
        grid=(GRID,),
        in_specs=[_DEG_SPEC, _Z_SPEC, _row_spec(H), _full((H, OUT)),
                  _full((1, OUT))],
        out_specs=_row_spec(OUT),
        out_shape=jax.ShapeDtypeStruct((NP, OUT), jnp.float32),
    )(degp, z, y3, w3, b3)


def kernel(x, edge_index, W1, b1, W2, b2, W_ih_f, W_hh_f, b_ih_f, b_hh_f,
           W_ih_b, W_hh_b, b_ih_b, b_hh_b, att_W, att_b, W3, b3):
    f32 = jnp.float32
    pad = DUMMY + (jnp.arange(EP - E, dtype=jnp.int32) % (NP - N))
    src3 = jnp.concatenate([edge_index[0], pad]).reshape(NW, KCH, CH)
    dst3 = jnp.concatenate([edge_index[1], pad]).reshape(NW, KCH, CH)
    ones_ch = jnp.ones((CH,), f32)
    zeros_r = jnp.zeros((RPT,), f32)
    zeros_rh = jnp.zeros((RPT, H), f32)

    def gate_args(wih, whh, bih, bhh):
        b = (bih + bhh)
        out = []
        for g in range(4):
            out.append(wih[g * LH:(g + 1) * LH].T)
        for g in range(4):
            out.append(whh[g * LH:(g + 1) * LH].T)
        for g in range(4):
            out.append(b[g * LH:(g + 1) * LH].reshape(1, LH))
        return out

    xp = jnp.zeros((NP, D), f32).at[:N].set(x)
    degp = _sc_degree(dst3, ones_ch, zeros_r)

    def scat(yp):
        return _sc_scatter(yp, src3, dst3, zeros_rh)

    y1 = _call_first(degp, xp, W1)
    z1 = scat(y1)
    x1, y2 = _call_conv_epilogue(degp, z1, y1, b1.reshape(1, H), W2)
    z2 = scat(y2)
    y3 = _call_jk(degp, z2, y2, b2.reshape(1, H), x1,
                  *gate_args(W_ih_f, W_hh_f, b_ih_f, b_hh_f),
                  *gate_args(W_ih_b, W_hh_b, b_ih_b, b_hh_b),
                  att_W[:LH, 0].reshape(1, LH), att_W[LH:, 0].reshape(1, LH))
    z3 = scat(y3)
    outp = _call_final(degp, z3, y3, W3, b3.reshape(1, OUT))
    return outp[:N]


# speedup vs baseline: 54.9721x; 1.0247x over previous
import functools

import jax
import jax.numpy as jnp
from jax import lax
from jax.experimental import pallas as pl
from jax.experimental.pallas import tpu as pltpu
from jax.experimental.pallas import tpu_sc as plsc

N = 10000
D = 128
H = 16
LH = 32
OUT = 40
E = 320000

NC = 2
NS = 16
NW = NC * NS
CH = 128
KCH = 80
EPT = KCH * CH
EP = NW * EPT
NP = 10240
RPT = NP // NS
DUMMY = N

_mesh = plsc.VectorSubcoreMesh(core_axis_name="c", subcore_axis_name="s")
_sc_params = pltpu.CompilerParams(use_tc_tiling_on_sc=False)


@functools.partial(
    pl.kernel, mesh=_mesh, compiler_params=_sc_params,
    out_type=jax.ShapeDtypeStruct((NC, NP), jnp.float32),
    scratch_types=[
        pltpu.VMEM((KCH, CH), jnp.int32),
        pltpu.VMEM((CH,), jnp.float32),
        pltpu.VMEM((RPT,), jnp.float32),
        pltpu.VMEM_SHARED((NP,), jnp.float32),
        pltpu.SemaphoreType.DMA,
    ],
)
def _sc_degree(dst3_hbm, ones_hbm, zeros_hbm, out_hbm,
               dst_v, ones_v, zrow_v, acc_sh, sem):
    cid = lax.axis_index("c")
    sid = lax.axis_index("s")
    wid = sid * NC + cid
    pltpu.sync_copy(zeros_hbm, zrow_v)
    pltpu.sync_copy(zrow_v, acc_sh.at[pl.ds(sid * RPT, RPT)])
    pltpu.sync_copy(ones_hbm, ones_v)
    pltpu.sync_copy(dst3_hbm.at[wid], dst_v)
    plsc.subcore_barrier()

    def body(j, carry):
        pltpu.async_copy(ones_v, acc_sh.at[dst_v.at[j]], sem, add=True)
        return carry

    lax.fori_loop(0, KCH, body, 0)

    def drain(j, carry):
        pltpu.make_async_copy(ones_v, acc_sh.at[dst_v.at[j]], sem).wait()
        return carry

    lax.fori_loop(0, KCH, drain, 0)
    plsc.subcore_barrier()
    pltpu.sync_copy(acc_sh.at[pl.ds(sid * RPT, RPT)],
                    out_hbm.at[cid, pl.ds(sid * RPT, RPT)])


NB = 8
NGRP = KCH // NB


@functools.partial(
    pl.kernel, mesh=_mesh, compiler_params=_sc_params,
    out_type=jax.ShapeDtypeStruct((NC, NP, H), jnp.float32),
    scratch_types=[
        pltpu.VMEM((KCH, CH), jnp.int32),
        pltpu.VMEM((KCH, CH), jnp.int32),
        pltpu.VMEM((NB, CH, H), jnp.float32),
        pltpu.VMEM((RPT, H), jnp.float32),
        pltpu.VMEM_SHARED((NP, H), jnp.float32),
        pltpu.SemaphoreType.DMA((NB,)),
        pltpu.SemaphoreType.DMA((NB,)),
    ],
)
def _sc_scatter(y_hbm, src3_hbm, dst3_hbm, zeros_hbm, out_hbm,
                src_v, dst_v, rows_v, zrow_v, acc_sh, gsem, ssem):
    cid = lax.axis_index("c")
    sid = lax.axis_index("s")
    wid = sid * NC + cid
    pltpu.sync_copy(zeros_hbm, zrow_v)
    pltpu.sync_copy(zrow_v, acc_sh.at[pl.ds(sid * RPT, RPT)])
    pltpu.sync_copy(src3_hbm.at[wid], src_v)
    pltpu.sync_copy(dst3_hbm.at[wid], dst_v)
    plsc.subcore_barrier()

    def _gather_start(j, b):
        pltpu.async_copy(y_hbm.at[src_v.at[j]], rows_v.at[b], gsem.at[b])

    def _gather_wait(j, b):
        pltpu.make_async_copy(y_hbm.at[src_v.at[j]], rows_v.at[b],
                              gsem.at[b]).wait()

    def _scatter_start(j, b):
        pltpu.async_copy(rows_v.at[b], acc_sh.at[dst_v.at[j]], ssem.at[b],
                         add=True)

    def _scatter_wait(j, b):
        pltpu.make_async_copy(rows_v.at[b], acc_sh.at[dst_v.at[j]],
                              ssem.at[b]).wait()

    for b in range(NB):
        _gather_start(b, b)

    def group(g, carry):
        for b in range(NB):
            j = g * NB + b
            _gather_wait(j, b)
            _scatter_start(j, b)

            @pl.when(g < NGRP - 1)
            def _():
                _scatter_wait(j, b)
                _gather_start(j + NB, b)

        return carry

    lax.fori_loop(0, NGRP, group, 0)
    for b in range(NB):
        _scatter_wait((NGRP - 1) * NB + b, b)
    plsc.subcore_barrier()
    pltpu.sync_copy(acc_sh.at[pl.ds(sid * RPT, RPT)],
                    out_hbm.at[cid, pl.ds(sid * RPT, RPT)])


BR = 2048
GRID = NP // BR


def _dinv_of(deg_ref):
    return lax.rsqrt(deg_ref[0, :] + deg_ref[1, :] + 1.0)[:, None]


def _tc_first(deg_ref, x_ref, w_ref, y_ref):
    xw = jnp.dot(x_ref[...], w_ref[...], preferred_element_type=jnp.float32)
    y_ref[...] = xw * _dinv_of(deg_ref)


def _tc_conv_epilogue(deg_ref, z_ref, y_ref, b_ref, w_ref, x1_ref, y2_ref):
    dinv = _dinv_of(deg_ref)
    x1 = jnp.maximum((z_ref[0] + z_ref[1] + y_ref[...]) * dinv + b_ref[0, :],
                     0.0)
    x1_ref[...] = x1
    y2_ref[...] = jnp.dot(x1, w_ref[...],
                          preferred_element_type=jnp.float32) * dinv


def _sig(v):
    return jax.nn.sigmoid(v)


def _dot(a, b_ref):
    return jnp.dot(a, b_ref[...], preferred_element_type=jnp.float32)


def _lstm_dir(xa, xb, w, y3_ref=None):
    wxi, wxf, wxg, wxo, whi, whf, whg, who, bi, bf, bg, bo = w
    gi = _sig(_dot(xa, wxi) + bi[0, :])
    gg = jnp.tanh(_dot(xa, wxg) + bg[0, :])
    go = _sig(_dot(xa, wxo) + bo[0, :])
    c1 = gi * gg
    h1 = go * jnp.tanh(c1)
    gi = _sig(_dot(xb, wxi) + _dot(h1, whi) + bi[0, :])
    gf = _sig(_dot(xb, wxf) + _dot(h1, whf) + bf[0, :])
    gg = jnp.tanh(_dot(xb, wxg) + _dot(h1, whg) + bg[0, :])
    go = _sig(_dot(xb, wxo) + _dot(h1, who) + bo[0, :])
    c2 = gf * c1 + gi * gg
    h2 = go * jnp.tanh(c2)
    return h1, h2


def _tc_jk(deg_ref, z_ref, y_ref, b2_ref, x1_ref,
           wxi_f, wxf_f, wxg_f, wxo_f, whi_f, whf_f, whg_f, who_f,
           bi_f, bf_f, bg_f, bo_f,
           wxi_b, wxf_b, wxg_b, wxo_b, whi_b, whf_b, whg_b, who_b,
           bi_b, bf_b, bg_b, bo_b,
           aa_ref, ab_ref, y3_ref):
    dinv = _dinv_of(deg_ref)
    x1 = x1_ref[...]
    x2 = jnp.maximum((z_ref[0] + z_ref[1] + y_ref[...]) * dinv + b2_ref[0, :],
                     0.0)
    h1f, h2f = _lstm_dir(x1, x2, (wxi_f, wxf_f, wxg_f, wxo_f,
                                  whi_f, whf_f, whg_f, who_f,
                                  bi_f, bf_f, bg_f, bo_f))
    h2b, h1b = _lstm_dir(x2, x1, (wxi_b, wxf_b, wxg_b, wxo_b,
                                  whi_b, whf_b, whg_b, who_b,
                                  bi_b, bf_b, bg_b, bo_b))
    delta = jnp.sum((h1f - h2f) * aa_ref[0, :] + (h1b - h2b) * ab_ref[0, :],
                    axis=1)
    w0 = _sig(delta)[:, None]
    jk = w0 * x1 + (1.0 - w0) * x2
    y3_ref[...] = jk * dinv


def _tc_final(deg_ref, z_ref, y_ref, w_ref, b_ref, out_ref):
    dinv = _dinv_of(deg_ref)
    prop = (z_ref[0] + z_ref[1] + y_ref[...]) * dinv
    logits = jnp.dot(prop, w_ref[...],
                     preferred_element_type=jnp.float32) + b_ref[0, :]
    m = jnp.max(logits, axis=1, keepdims=True)
    s = jnp.log(jnp.sum(jnp.exp(logits - m), axis=1, keepdims=True))
    out_ref[...] = logits - m - s


def _row_spec(width):
    return pl.BlockSpec((BR, width), lambda i: (i, 0))


_DEG_SPEC = pl.BlockSpec((2, BR), lambda i: (0, i))
_Z_SPEC = pl.BlockSpec((2, BR, H), lambda i: (0, i, 0))


def _full(shape):
    return pl.BlockSpec(shape, lambda i: tuple(0 for _ in shape))


def _call_first(degp, xp, w1):
    return pl.pallas_call(
        _tc_first,
        grid=(GRID,),
        in_specs=[_DEG_SPEC, _row_spec(D), _full((D, H))],
        out_specs=_row_spec(H),
        out_shape=jax.ShapeDtypeStruct((NP, H), jnp.float32),
    )(degp, xp, w1)


def _call_conv_epilogue(degp, z, y, b, w):
    return pl.pallas_call(
        _tc_conv_epilogue,
        grid=(GRID,),
        in_specs=[_DEG_SPEC, _Z_SPEC, _row_spec(H), _full((1, H)),
                  _full((H, H))],
        out_specs=[_row_spec(H), _row_spec(H)],
        out_shape=[jax.ShapeDtypeStruct((NP, H), jnp.float32),
                   jax.ShapeDtypeStruct((NP, H), jnp.float32)],
    )(degp, z, y, b, w)


def _call_jk(degp, z, y2, b2, x1, *lstm_args):
    wspecs = ([_full((H, LH))] * 4 + [_full((LH, LH))] * 4 +
              [_full((1, LH))] * 4) * 2 + [_full((1, LH))] * 2
    return pl.pallas_call(
        _tc_jk,
        grid=(GRID,),
        in_specs=[_DEG_SPEC, _Z_SPEC, _row_spec(H), _full((1, H)),
                  _row_spec(H)] + wspecs,
        out_specs=_row_spec(H),
        out_shape=jax.ShapeDtypeStruct((NP, H), jnp.float32),
    )(degp, z, y2, b2, x1, *lstm_args)


def _call_final(degp, z, y3, w3, b3):
    return pl.pallas_call(
        _tc_final,
        grid=(GRID,),
        in_specs=[_DEG_SPEC, _Z_SPEC, _row_spec(H), _full((H, OUT)),
                  _full((1, OUT))],
        out_specs=_row_spec(OUT),
        out_shape=jax.ShapeDtypeStruct((NP, OUT), jnp.float32),
    )(degp, z, y3, w3, b3)


def kernel(x, edge_index, W1, b1, W2, b2, W_ih_f, W_hh_f, b_ih_f, b_hh_f,
           W_ih_b, W_hh_b, b_ih_b, b_hh_b, att_W, att_b, W3, b3):
    f32 = jnp.float32
    pad = DUMMY + (jnp.arange(EP - E, dtype=jnp.int32) % (NP - N))
    src3 = jnp.concatenate([edge_index[0], pad]).reshape(NW, KCH, CH)
    dst3 = jnp.concatenate([edge_index[1], pad]).reshape(NW, KCH, CH)
    ones_ch = jnp.ones((CH,), f32)
    zeros_r = jnp.zeros((RPT,), f32)
    zeros_rh = jnp.zeros((RPT, H), f32)

    def gate_args(wih, whh, bih, bhh):
        b = (bih + bhh)
        out = []
        for g in range(4):
            out.append(wih[g * LH:(g + 1) * LH].T)
        for g in range(4):
            out.append(whh[g * LH:(g + 1) * LH].T)
        for g in range(4):
            out.append(b[g * LH:(g + 1) * LH].reshape(1, LH))
        return out

    xp = jnp.zeros((NP, D), f32).at[:N].set(x)
    degp = _sc_degree(dst3, ones_ch, zeros_r)

    def scat(yp):
        return _sc_scatter(yp, src3, dst3, zeros_rh)

    y1 = _call_first(degp, xp, W1)
    z1 = scat(y1)
    x1, y2 = _call_conv_epilogue(degp, z1, y1, b1.reshape(1, H), W2)
    z2 = scat(y2)
    y3 = _call_jk(degp, z2, y2, b2.reshape(1, H), x1,
                  *gate_args(W_ih_f, W_hh_f, b_ih_f, b_hh_f),
                  *gate_args(W_ih_b, W_hh_b, b_ih_b, b_hh_b),
                  att_W[:LH, 0].reshape(1, LH), att_W[LH:, 0].reshape(1, LH))
    z3 = scat(y3)
    outp = _call_final(degp, z3, y3, W3, b3.reshape(1, OUT))
    return outp[:N]


# speedup vs baseline: 55.8591x; 1.0161x over previous
import functools

import jax
import jax.numpy as jnp
from jax import lax
from jax.experimental import pallas as pl
from jax.experimental.pallas import tpu as pltpu
from jax.experimental.pallas import tpu_sc as plsc

N = 10000
D = 128
H = 16
LH = 32
OUT = 40
E = 320000

NC = 2
NS = 16
NW = NC * NS
CH = 128
KCH = 80
EPT = KCH * CH
EP = NW * EPT
NP = 10240
RPT = NP // NS
DUMMY = N

_mesh = plsc.VectorSubcoreMesh(core_axis_name="c", subcore_axis_name="s")
_sc_params = pltpu.CompilerParams(use_tc_tiling_on_sc=False)


RKCH = 78
XTRA = NW * RKCH
DUMTOP = NP - N


@functools.partial(
    pl.kernel, mesh=_mesh,
    compiler_params=pltpu.CompilerParams(use_tc_tiling_on_sc=True),
    out_type=[jax.ShapeDtypeStruct((NW, KCH, CH), jnp.int32),
              jax.ShapeDtypeStruct((NW, KCH, CH), jnp.int32)],
    scratch_types=[
        pltpu.VMEM((KCH, 2, CH), jnp.int32),
        pltpu.VMEM((KCH, CH), jnp.int32),
        pltpu.VMEM((KCH, CH), jnp.int32),
        pltpu.SemaphoreType.DMA,
    ],
)
def _sc_stage(edge_hbm, src3_hbm, dst3_hbm, e_v, src_v, dst_v, sem):
    cid = lax.axis_index("c")
    sid = lax.axis_index("s")
    wid = sid * NC + cid
    base = wid * RKCH * CH
    lane = lax.iota(jnp.int32, 16)
    for r in (RKCH, RKCH + 1):
        for k in range(CH // 16):
            fill = DUMMY + (lane + 16 * k + 128 * (r - RKCH)) % DUMTOP
            src_v[r, pl.ds(16 * k, 16)] = fill
            dst_v[r, pl.ds(16 * k, 16)] = fill

    def load(j, carry):
        pltpu.async_copy(edge_hbm.at[:, pl.ds(base + j * CH, CH)],
                         e_v.at[j], sem)
        return carry

    lax.fori_loop(0, RKCH, load, 0)

    @pl.when(wid < 4)
    def _():
        pltpu.async_copy(edge_hbm.at[:, pl.ds(XTRA * CH + wid * CH, CH)],
                         e_v.at[RKCH], sem)

    def drain(j, carry):
        pltpu.make_async_copy(edge_hbm.at[:, pl.ds(base + j * CH, CH)],
                              e_v.at[j], sem).wait()
        return carry

    lax.fori_loop(0, RKCH, drain, 0)

    @pl.when(wid < 4)
    def _():
        pltpu.make_async_copy(edge_hbm.at[:, pl.ds(XTRA * CH + wid * CH, CH)],
                              e_v.at[RKCH], sem).wait()

    def deint(j, carry):
        for k in range(CH // 16):
            src_v[j, pl.ds(16 * k, 16)] = e_v[j, 0, pl.ds(16 * k, 16)]
            dst_v[j, pl.ds(16 * k, 16)] = e_v[j, 1, pl.ds(16 * k, 16)]
        return carry

    lax.fori_loop(0, RKCH, deint, 0)

    @pl.when(wid < 4)
    def _():
        for k in range(CH // 16):
            src_v[RKCH, pl.ds(16 * k, 16)] = e_v[RKCH, 0, pl.ds(16 * k, 16)]
            dst_v[RKCH, pl.ds(16 * k, 16)] = e_v[RKCH, 1, pl.ds(16 * k, 16)]

    pltpu.sync_copy(src_v, src3_hbm.at[wid])
    pltpu.sync_copy(dst_v, dst3_hbm.at[wid])

@functools.partial(
    pl.kernel, mesh=_mesh, compiler_params=_sc_params,
    out_type=jax.ShapeDtypeStruct((NC, NP), jnp.float32),
    scratch_types=[
        pltpu.VMEM((KCH, CH), jnp.int32),
        pltpu.VMEM((CH,), jnp.float32),
        pltpu.VMEM((RPT,), jnp.float32),
        pltpu.VMEM_SHARED((NP,), jnp.float32),
        pltpu.SemaphoreType.DMA,
    ],
)
def _sc_degree(dst3_hbm, ones_hbm, zeros_hbm, out_hbm,
               dst_v, ones_v, zrow_v, acc_sh, sem):
    cid = lax.axis_index("c")
    sid = lax.axis_index("s")
    wid = sid * NC + cid
    pltpu.sync_copy(zeros_hbm, zrow_v)
    pltpu.sync_copy(zrow_v, acc_sh.at[pl.ds(sid * RPT, RPT)])
    pltpu.sync_copy(ones_hbm, ones_v)
    pltpu.sync_copy(dst3_hbm.at[wid], dst_v)
    plsc.subcore_barrier()

    def body(j, carry):
        pltpu.async_copy(ones_v, acc_sh.at[dst_v.at[j]], sem, add=True)
        return carry

    lax.fori_loop(0, KCH, body, 0)

    def drain(j, carry):
        pltpu.make_async_copy(ones_v, acc_sh.at[dst_v.at[j]], sem).wait()
        return carry

    lax.fori_loop(0, KCH, drain, 0)
    plsc.subcore_barrier()
    pltpu.sync_copy(acc_sh.at[pl.ds(sid * RPT, RPT)],
                    out_hbm.at[cid, pl.ds(sid * RPT, RPT)])


NB = 8
NGRP = KCH // NB


@functools.partial(
    pl.kernel, mesh=_mesh, compiler_params=_sc_params,
    out_type=jax.ShapeDtypeStruct((NC, NP, H), jnp.float32),
    scratch_types=[
        pltpu.VMEM((KCH, CH), jnp.int32),
        pltpu.VMEM((KCH, CH), jnp.int32),
        pltpu.VMEM((NB, CH, H), jnp.float32),
        pltpu.VMEM((RPT, H), jnp.float32),
        pltpu.VMEM_SHARED((NP, H), jnp.float32),
        pltpu.SemaphoreType.DMA((NB,)),
        pltpu.SemaphoreType.DMA((NB,)),
    ],
)
def _sc_scatter(y_hbm, src3_hbm, dst3_hbm, zeros_hbm, out_hbm,
                src_v, dst_v, rows_v, zrow_v, acc_sh, gsem, ssem):
    cid = lax.axis_index("c")
    sid = lax.axis_index("s")
    wid = sid * NC + cid
    pltpu.sync_copy(zeros_hbm, zrow_v)
    pltpu.sync_copy(zrow_v, acc_sh.at[pl.ds(sid * RPT, RPT)])
    pltpu.sync_copy(src3_hbm.at[wid], src_v)
    pltpu.sync_copy(dst3_hbm.at[wid], dst_v)
    plsc.subcore_barrier()

    def _gather_start(j, b):
        pltpu.async_copy(y_hbm.at[src_v.at[j]], rows_v.at[b], gsem.at[b])

    def _gather_wait(j, b):
        pltpu.make_async_copy(y_hbm.at[src_v.at[j]], rows_v.at[b],
                              gsem.at[b]).wait()

    def _scatter_start(j, b):
        pltpu.async_copy(rows_v.at[b], acc_sh.at[dst_v.at[j]], ssem.at[b],
                         add=True)

    def _scatter_wait(j, b):
        pltpu.make_async_copy(rows_v.at[b], acc_sh.at[dst_v.at[j]],
                              ssem.at[b]).wait()

    for b in range(NB):
        _gather_start(b, b)

    def group(g, carry):
        for b in range(NB):
            j = g * NB + b
            _gather_wait(j, b)
            _scatter_start(j, b)

            @pl.when(g < NGRP - 1)
            def _():
                _scatter_wait(j, b)
                _gather_start(j + NB, b)

        return carry

    lax.fori_loop(0, NGRP, group, 0)
    for b in range(NB):
        _scatter_wait((NGRP - 1) * NB + b, b)
    plsc.subcore_barrier()
    pltpu.sync_copy(acc_sh.at[pl.ds(sid * RPT, RPT)],
                    out_hbm.at[cid, pl.ds(sid * RPT, RPT)])


BR = 2048
GRID = NP // BR


def _dinv_of(deg_ref):
    return lax.rsqrt(deg_ref[0, :] + deg_ref[1, :] + 1.0)[:, None]


def _tc_first(deg_ref, x_ref, w_ref, y_ref):
    xw = jnp.dot(x_ref[...], w_ref[...], preferred_element_type=jnp.float32)
    y_ref[...] = xw * _dinv_of(deg_ref)


def _tc_conv_epilogue(deg_ref, z_ref, y_ref, b_ref, w_ref, x1_ref, y2_ref):
    dinv = _dinv_of(deg_ref)
    x1 = jnp.maximum((z_ref[0] + z_ref[1] + y_ref[...]) * dinv + b_ref[0, :],
                     0.0)
    x1_ref[...] = x1
    y2_ref[...] = jnp.dot(x1, w_ref[...],
                          preferred_element_type=jnp.float32) * dinv


def _sig(v):
    return jax.nn.sigmoid(v)


def _dot(a, b_ref):
    return jnp.dot(a, b_ref[...], preferred_element_type=jnp.float32)


def _lstm_dir(xa, xb, w, y3_ref=None):
    wxi, wxf, wxg, wxo, whi, whf, whg, who, bi, bf, bg, bo = w
    gi = _sig(_dot(xa, wxi) + bi[0, :])
    gg = jnp.tanh(_dot(xa, wxg) + bg[0, :])
    go = _sig(_dot(xa, wxo) + bo[0, :])
    c1 = gi * gg
    h1 = go * jnp.tanh(c1)
    gi = _sig(_dot(xb, wxi) + _dot(h1, whi) + bi[0, :])
    gf = _sig(_dot(xb, wxf) + _dot(h1, whf) + bf[0, :])
    gg = jnp.tanh(_dot(xb, wxg) + _dot(h1, whg) + bg[0, :])
    go = _sig(_dot(xb, wxo) + _dot(h1, who) + bo[0, :])
    c2 = gf * c1 + gi * gg
    h2 = go * jnp.tanh(c2)
    return h1, h2


def _tc_jk(deg_ref, z_ref, y_ref, b2_ref, x1_ref,
           wxi_f, wxf_f, wxg_f, wxo_f, whi_f, whf_f, whg_f, who_f,
           bi_f, bf_f, bg_f, bo_f,
           wxi_b, wxf_b, wxg_b, wxo_b, whi_b, whf_b, whg_b, who_b,
           bi_b, bf_b, bg_b, bo_b,
           aa_ref, ab_ref, y3_ref):
    dinv = _dinv_of(deg_ref)
    x1 = x1_ref[...]
    x2 = jnp.maximum((z_ref[0] + z_ref[1] + y_ref[...]) * dinv + b2_ref[0, :],
                     0.0)
    h1f, h2f = _lstm_dir(x1, x2, (wxi_f, wxf_f, wxg_f, wxo_f,
                                  whi_f, whf_f, whg_f, who_f,
                                  bi_f, bf_f, bg_f, bo_f))
    h2b, h1b = _lstm_dir(x2, x1, (wxi_b, wxf_b, wxg_b, wxo_b,
                                  whi_b, whf_b, whg_b, who_b,
                                  bi_b, bf_b, bg_b, bo_b))
    delta = jnp.sum((h1f - h2f) * aa_ref[0, :] + (h1b - h2b) * ab_ref[0, :],
                    axis=1)
    w0 = _sig(delta)[:, None]
    jk = w0 * x1 + (1.0 - w0) * x2
    y3_ref[...] = jk * dinv


def _tc_final(deg_ref, z_ref, y_ref, w_ref, b_ref, out_ref):
    dinv = _dinv_of(deg_ref)
    prop = (z_ref[0] + z_ref[1] + y_ref[...]) * dinv
    logits = jnp.dot(prop, w_ref[...],
                     preferred_element_type=jnp.float32) + b_ref[0, :]
    m = jnp.max(logits, axis=1, keepdims=True)
    s = jnp.log(jnp.sum(jnp.exp(logits - m), axis=1, keepdims=True))
    out_ref[...] = logits - m - s


def _row_spec(width):
    return pl.BlockSpec((BR, width), lambda i: (i, 0))


_DEG_SPEC = pl.BlockSpec((2, BR), lambda i: (0, i))
_Z_SPEC = pl.BlockSpec((2, BR, H), lambda i: (0, i, 0))


def _full(shape):
    return pl.BlockSpec(shape, lambda i: tuple(0 for _ in shape))


def _call_first(degp, xp, w1):
    return pl.pallas_call(
        _tc_first,
        grid=(GRID,),
        in_specs=[_DEG_SPEC, _row_spec(D), _full((D, H))],
        out_specs=_row_spec(H),
        out_shape=jax.ShapeDtypeStruct((NP, H), jnp.float32),
    )(degp, xp, w1)


def _call_conv_epilogue(degp, z, y, b, w):
    return pl.pallas_call(
        _tc_conv_epilogue,
        grid=(GRID,),
        in_specs=[_DEG_SPEC, _Z_SPEC, _row_spec(H), _full((1, H)),
                  _full((H, H))],
        out_specs=[_row_spec(H), _row_spec(H)],
        out_shape=[jax.ShapeDtypeStruct((NP, H), jnp.float32),
                   jax.ShapeDtypeStruct((NP, H), jnp.float32)],
    )(degp, z, y, b, w)


def _call_jk(degp, z, y2, b2, x1, *lstm_args):
    wspecs = ([_full((H, LH))] * 4 + [_full((LH, LH))] * 4 +
              [_full((1, LH))] * 4) * 2 + [_full((1, LH))] * 2
    return pl.pallas_call(
        _tc_jk,
        grid=(GRID,),
        in_specs=[_DEG_SPEC, _Z_SPEC, _row_spec(H), _full((1, H)),
                  _row_spec(H)] + wspecs,
        out_specs=_row_spec(H),
        out_shape=jax.ShapeDtypeStruct((NP, H), jnp.float32),
    )(degp, z, y2, b2, x1, *lstm_args)


def _call_final(degp, z, y3, w3, b3):
    return pl.pallas_call(
        _tc_final,
        grid=(GRID,),
        in_specs=[_DEG_SPEC, _Z_SPEC, _row_spec(H), _full((H, OUT)),
                  _full((1, OUT))],
        out_specs=_row_spec(OUT),
        out_shape=jax.ShapeDtypeStruct((NP, OUT), jnp.float32),
    )(degp, z, y3, w3, b3)


def kernel(x, edge_index, W1, b1, W2, b2, W_ih_f, W_hh_f, b_ih_f, b_hh_f,
           W_ih_b, W_hh_b, b_ih_b, b_hh_b, att_W, att_b, W3, b3):
    f32 = jnp.float32
    src3, dst3 = _sc_stage(edge_index)
    ones_ch = jnp.ones((CH,), f32)
    zeros_r = jnp.zeros((RPT,), f32)
    zeros_rh = jnp.zeros((RPT, H), f32)

    def gate_args(wih, whh, bih, bhh):
        b = (bih + bhh)
        out = []
        for g in range(4):
            out.append(wih[g * LH:(g + 1) * LH].T)
        for g in range(4):
            out.append(whh[g * LH:(g + 1) * LH].T)
        for g in range(4):
            out.append(b[g * LH:(g + 1) * LH].reshape(1, LH))
        return out

    xp = jnp.zeros((NP, D), f32).at[:N].set(x)
    degp = _sc_degree(dst3, ones_ch, zeros_r)

    def scat(yp):
        return _sc_scatter(yp, src3, dst3, zeros_rh)

    y1 = _call_first(degp, xp, W1)
    z1 = scat(y1)
    x1, y2 = _call_conv_epilogue(degp, z1, y1, b1.reshape(1, H), W2)
    z2 = scat(y2)
    y3 = _call_jk(degp, z2, y2, b2.reshape(1, H), x1,
                  *gate_args(W_ih_f, W_hh_f, b_ih_f, b_hh_f),
                  *gate_args(W_ih_b, W_hh_b, b_ih_b, b_hh_b),
                  att_W[:LH, 0].reshape(1, LH), att_W[LH:, 0].reshape(1, LH))
    z3 = scat(y3)
    outp = _call_final(degp, z3, y3, W3, b3.reshape(1, OUT))
    return outp[:N]
